# Initial kernel scaffold; baseline (speedup 1.0000x reference)
import jax
import jax.numpy as jnp
from jax.experimental import pallas as pl


def kernel(embeds, edge_index):
    raise NotImplementedError("write your pallas kernel here")


# speedup vs baseline: 4.6102x; 4.6102x over previous
import functools

import jax
import jax.numpy as jnp
from jax import lax
from jax.experimental import pallas as pl
from jax.experimental.pallas import tpu as pltpu
from jax.experimental.pallas import tpu_sc as plsc

_N = 10000
_E = 320000
_D = 128
_K = 512
_NPAD = 10240
_NC = 2
_NS = 16
_NW = _NC * _NS
_EPT = _E // _NW
_B = 80
_NCH = _EPT // _B
_RPS = _NPAD // _NS
_W = 16


def _spmm_body(edge_hbm, x_hbm, dtab_hbm, zx_hbm, zn_hbm,
               xp_hbm, wp_hbm,
               dst_v, src_v, idx_v, rows_v, dtab_v, dacc_v, accx_sh):
  cid = lax.axis_index("c")
  sid = lax.axis_index("s")
  wid = sid * _NC + cid
  row0 = sid * _RPS
  lanes = lax.iota(jnp.int32, 16)

  def fill_idx(k):
    for j in range(_B // 16):
      idx_v[pl.ds(j * 16, 16)] = row0 + k * _B + j * 16 + lanes

  pltpu.sync_copy(zx_hbm, rows_v)
  pltpu.sync_copy(dtab_hbm, dtab_v)
  pltpu.sync_copy(zn_hbm, dacc_v)
  for k in range(_RPS // _B):
    fill_idx(k)
    pltpu.sync_copy(rows_v, accx_sh.at[idx_v])
  plsc.subcore_barrier()

  e0 = wid * _EPT

  def chunk(c, carry):
    base = e0 + c * _B
    pltpu.sync_copy(edge_hbm.at[pl.ds(base, _B)], dst_v)
    pltpu.sync_copy(edge_hbm.at[pl.ds(_E + base, _B)], src_v)
    pltpu.sync_copy(x_hbm.at[src_v], rows_v)
    for j in range(_B // 16):
      sidx = src_v[pl.ds(j * 16, 16)]
      didx = dst_v[pl.ds(j * 16, 16)]
      dvals = plsc.load_gather(dtab_v, [sidx])
      plsc.addupdate_scatter(dacc_v, [didx], dvals)
    pltpu.sync_copy(rows_v, accx_sh.at[dst_v], add=True)
    return carry

  lax.fori_loop(0, _NCH, chunk, 0)
  plsc.subcore_barrier()

  for k in range(_RPS // _B):
    fill_idx(k)
    pltpu.sync_copy(accx_sh.at[idx_v], rows_v)
    pltpu.sync_copy(rows_v, xp_hbm.at[cid, pl.ds(row0 + k * _B, _B)])
  pltpu.sync_copy(dacc_v, wp_hbm.at[pl.ds(wid * _NPAD, _NPAD)])


@functools.cache
def _spmm_kernel():
  mesh = plsc.VectorSubcoreMesh(
      core_axis_name="c", subcore_axis_name="s",
      num_cores=_NC, num_subcores=_NS)
  return pl.kernel(
      _spmm_body,
      out_type=(
          jax.ShapeDtypeStruct((_NC, _NPAD, _D), jnp.float32),
          jax.ShapeDtypeStruct((_NW * _NPAD,), jnp.float32),
      ),
      mesh=mesh,
      compiler_params=pltpu.CompilerParams(needs_layout_passes=False),
      scratch_types=[
          pltpu.VMEM((_B,), jnp.int32),
          pltpu.VMEM((_B,), jnp.int32),
          pltpu.VMEM((_B,), jnp.int32),
          pltpu.VMEM((_B, _D), jnp.float32),
          pltpu.VMEM((_NPAD,), jnp.float32),
          pltpu.VMEM((_NPAD,), jnp.float32),
          pltpu.VMEM_SHARED((_NPAD, _D), jnp.float32),
      ],
  )


def _spmm(edge_1d, x, dtab, zx, zw):
  return _spmm_kernel()(edge_1d, x, dtab, zx, zw)


def _combine_body(h1p_ref, degp_ref, embp_ref, fst_ref, deg_ref):
  fst_ref[...] = h1p_ref[0] + h1p_ref[1] - embp_ref[...]
  deg_ref[...] = jnp.sum(degp_ref[...], axis=0)


def _combine(h1p, degp, embp):
  nblk = 16
  rb = _NPAD // nblk
  return pl.pallas_call(
      _combine_body,
      grid=(nblk,),
      in_specs=[
          pl.BlockSpec((_NC, rb, _D), lambda i: (0, i, 0)),
          pl.BlockSpec((_NW, rb, 1), lambda i: (0, i, 0)),
          pl.BlockSpec((rb, _D), lambda i: (i, 0)),
      ],
      out_specs=[
          pl.BlockSpec((rb, _D), lambda i: (i, 0)),
          pl.BlockSpec((rb, 1), lambda i: (i, 0)),
      ],
      out_shape=[
          jax.ShapeDtypeStruct((_NPAD, _D), jnp.float32),
          jax.ShapeDtypeStruct((_NPAD, 1), jnp.float32),
      ],
  )(h1p, degp, embp)


def _score_body(emb_ref, fst_ref, h2p_ref, deg_ref, d2p_ref, noise_ref,
                out_ref):
  emb = emb_ref[...]
  fst = fst_ref[...]
  h2 = h2p_ref[0] + h2p_ref[1]
  deg = deg_ref[...]
  d2 = jnp.sum(d2p_ref[...], axis=0)
  scd = (h2 - fst) - deg * emb
  vec = fst + scd
  scdn = (d2 - deg) - deg
  den = (deg + scdn) + jnp.float32(1e-8)
  r = vec / den
  nr = jnp.sqrt(jnp.sum(r * r, axis=1, keepdims=True))
  rn = r / jnp.maximum(nr, jnp.float32(1e-12))
  ne = jnp.sqrt(jnp.sum(emb * emb, axis=1, keepdims=True))
  en = emb / jnp.maximum(ne, jnp.float32(1e-12))
  s = jnp.sum(rn * en, axis=1, keepdims=True)
  out_ref[...] = jnp.log(jax.nn.sigmoid(s)) + noise_ref[...]


def _score(embeds, fst, h2p, deg2d, d2p, noise2d):
  nblk = 25
  rb = _N // nblk
  return pl.pallas_call(
      _score_body,
      grid=(nblk,),
      in_specs=[
          pl.BlockSpec((rb, _D), lambda i: (i, 0)),
          pl.BlockSpec((rb, _D), lambda i: (i, 0)),
          pl.BlockSpec((_NC, rb, _D), lambda i: (0, i, 0)),
          pl.BlockSpec((rb, 1), lambda i: (i, 0)),
          pl.BlockSpec((_NW, rb, 1), lambda i: (0, i, 0)),
          pl.BlockSpec((rb, 1), lambda i: (i, 0)),
      ],
      out_specs=pl.BlockSpec((rb, 1), lambda i: (i, 0)),
      out_shape=jax.ShapeDtypeStruct((_N, 1), jnp.float32),
  )(embeds, fst, h2p, deg2d, d2p, noise2d)


def _rank_body(sc_col_ref, sc_row_ref, seeds_ref):
  sc_row = sc_row_ref[...]
  irow = lax.broadcasted_iota(jnp.int32, (1, _N), 1)
  jb = 400

  def step(c, rank_row):
    sj = sc_col_ref[pl.ds(c * jb, jb), :]
    jcol = c * jb + lax.broadcasted_iota(jnp.int32, (jb, 1), 0)
    gt = (sj > sc_row).astype(jnp.int32)
    tie = ((sj == sc_row) & (jcol < irow)).astype(jnp.int32)
    return rank_row + jnp.sum(gt + tie, axis=0, keepdims=True)

  rank_row = lax.fori_loop(0, _N // jb, step,
                           jnp.zeros((1, _N), jnp.int32))
  pcol = lax.broadcasted_iota(jnp.int32, (_K, 1), 0)
  hit = (rank_row == pcol)
  seeds_ref[...] = jnp.sum(jnp.where(hit, irow, 0), axis=1, keepdims=True)


def _rank(sc_col, sc_row):
  return pl.pallas_call(
      _rank_body,
      out_shape=jax.ShapeDtypeStruct((_K, 1), jnp.int32),
  )(sc_col, sc_row)


def kernel(embeds, edge_index):
  f32 = jnp.float32
  embp = jnp.concatenate(
      [embeds, jnp.zeros((_NPAD - _N, _D), f32)], axis=0)
  ones1d = jnp.ones((_NPAD,), f32)
  zx = jnp.zeros((_B, _D), f32)
  zn = jnp.zeros((_NPAD,), f32)

  edge_1d = edge_index.reshape(2 * _E)
  h1p, degp = _spmm(edge_1d, embp, ones1d, zx, zn)
  fst, deg2d = _combine(h1p, degp.reshape(_NW, _NPAD, 1), embp)
  h2p, d2p = _spmm(edge_1d, fst, deg2d.reshape(_NPAD), zx, zn)
  d2p = d2p.reshape(_NW, _NPAD, 1)

  nk = jax.random.fold_in(jax.random.key(0), 123)
  u = jax.random.uniform(nk, (_N,), dtype=f32, minval=1e-12, maxval=1.0)
  noise = -jnp.log(-jnp.log(u))

  scores2d = _score(embeds, fst, h2p, deg2d, d2p, noise.reshape(_N, 1))
  seeds2d = _rank(scores2d, scores2d.reshape(1, _N))
  return (scores2d.reshape(_N), seeds2d.reshape(_K))


# speedup vs baseline: 5.7809x; 1.2539x over previous
import functools

import jax
import jax.numpy as jnp
from jax import lax
from jax.experimental import pallas as pl
from jax.experimental.pallas import tpu as pltpu
from jax.experimental.pallas import tpu_sc as plsc

_N = 10000
_E = 320000
_D = 128
_K = 512
_NPAD = 10240
_NC = 2
_NS = 16
_NW = _NC * _NS
_EPT = _E // _NW
_B = 80
_NCH = _EPT // _B
_RPS = _NPAD // _NS
_W = 16


def _spmm_body(edge_hbm, x_hbm, dtab_hbm, zx_hbm, zn_hbm,
               xp_hbm, wp_hbm,
               dst0_v, src0_v, dst1_v, src1_v, idx_v,
               rows0_v, rows1_v, dtab_v, dacc_v, accx_sh,
               gsem0, gsem1, ssem0, ssem1):
  cid = lax.axis_index("c")
  sid = lax.axis_index("s")
  wid = sid * _NC + cid
  row0 = sid * _RPS
  lanes = lax.iota(jnp.int32, 16)
  dst = (dst0_v, dst1_v)
  src = (src0_v, src1_v)
  rows = (rows0_v, rows1_v)
  gsem = (gsem0, gsem1)
  ssem = (ssem0, ssem1)

  def fill_idx(k):
    for j in range(_B // 16):
      idx_v[pl.ds(j * 16, 16)] = row0 + k * _B + j * 16 + lanes

  pltpu.sync_copy(zx_hbm, rows0_v)
  pltpu.sync_copy(dtab_hbm, dtab_v)
  pltpu.sync_copy(zn_hbm, dacc_v)
  for k in range(_RPS // _B):
    fill_idx(k)
    pltpu.sync_copy(rows0_v, accx_sh.at[idx_v])
  plsc.subcore_barrier()

  e0 = wid * _EPT

  def load_idx(c, p):
    base = e0 + c * _B
    pltpu.sync_copy(edge_hbm.at[pl.ds(base, _B)], dst[p])
    pltpu.sync_copy(edge_hbm.at[pl.ds(_E + base, _B)], src[p])

  def gather(p):
    return pltpu.make_async_copy(x_hbm.at[src[p]], rows[p], gsem[p])

  def scatter(p):
    return pltpu.make_async_copy(rows[p], accx_sh.at[dst[p]], ssem[p])

  def scalar_channel(p):
    for j in range(_B // 16):
      sidx = src[p][pl.ds(j * 16, 16)]
      didx = dst[p][pl.ds(j * 16, 16)]
      dvals = plsc.load_gather(dtab_v, [sidx])
      plsc.addupdate_scatter(dacc_v, [didx], dvals)

  def slot(c, p, q, first):
    if not first:
      scatter(q).wait()
    load_idx(c + 1, q)
    gather(p).wait()
    gather(q).start()
    scalar_channel(p)
    scatter(p).start(add=True)

  load_idx(0, 0)
  gather(0).start()
  slot(0, 0, 1, True)

  def pair(g, carry):
    slot(2 * g + 1, 1, 0, False)
    slot(2 * g + 2, 0, 1, False)
    return carry

  lax.fori_loop(0, (_NCH - 3) // 2, pair, 0)
  slot(_NCH - 2, 1, 0, False)
  gather(0).wait()
  scalar_channel(0)
  scatter(0).start(add=True)
  scatter(1).wait()
  scatter(0).wait()
  plsc.subcore_barrier()

  for k in range(_RPS // _B):
    fill_idx(k)
    pltpu.sync_copy(accx_sh.at[idx_v], rows0_v)
    pltpu.sync_copy(rows0_v, xp_hbm.at[cid, pl.ds(row0 + k * _B, _B)])
  pltpu.sync_copy(dacc_v, wp_hbm.at[pl.ds(wid * _NPAD, _NPAD)])


@functools.cache
def _spmm_kernel():
  mesh = plsc.VectorSubcoreMesh(
      core_axis_name="c", subcore_axis_name="s",
      num_cores=_NC, num_subcores=_NS)
  return pl.kernel(
      _spmm_body,
      out_type=(
          jax.ShapeDtypeStruct((_NC, _NPAD, _D), jnp.float32),
          jax.ShapeDtypeStruct((_NW * _NPAD,), jnp.float32),
      ),
      mesh=mesh,
      compiler_params=pltpu.CompilerParams(needs_layout_passes=False),
      scratch_types=[
          pltpu.VMEM((_B,), jnp.int32),
          pltpu.VMEM((_B,), jnp.int32),
          pltpu.VMEM((_B,), jnp.int32),
          pltpu.VMEM((_B,), jnp.int32),
          pltpu.VMEM((_B,), jnp.int32),
          pltpu.VMEM((_B, _D), jnp.float32),
          pltpu.VMEM((_B, _D), jnp.float32),
          pltpu.VMEM((_NPAD,), jnp.float32),
          pltpu.VMEM((_NPAD,), jnp.float32),
          pltpu.VMEM_SHARED((_NPAD, _D), jnp.float32),
          pltpu.SemaphoreType.DMA,
          pltpu.SemaphoreType.DMA,
          pltpu.SemaphoreType.DMA,
          pltpu.SemaphoreType.DMA,
      ],
  )


def _spmm(edge_1d, x, dtab, zx, zw):
  return _spmm_kernel()(edge_1d, x, dtab, zx, zw)


def _combine_body(h1p_ref, degp_ref, embp_ref, fst_ref, deg_ref):
  fst_ref[...] = h1p_ref[0] + h1p_ref[1] - embp_ref[...]
  deg_ref[...] = jnp.sum(degp_ref[...], axis=0)


def _combine(h1p, degp, embp):
  nblk = 16
  rb = _NPAD // nblk
  return pl.pallas_call(
      _combine_body,
      grid=(nblk,),
      in_specs=[
          pl.BlockSpec((_NC, rb, _D), lambda i: (0, i, 0)),
          pl.BlockSpec((_NW, rb, 1), lambda i: (0, i, 0)),
          pl.BlockSpec((rb, _D), lambda i: (i, 0)),
      ],
      out_specs=[
          pl.BlockSpec((rb, _D), lambda i: (i, 0)),
          pl.BlockSpec((rb, 1), lambda i: (i, 0)),
      ],
      out_shape=[
          jax.ShapeDtypeStruct((_NPAD, _D), jnp.float32),
          jax.ShapeDtypeStruct((_NPAD, 1), jnp.float32),
      ],
  )(h1p, degp, embp)


def _score_body(emb_ref, fst_ref, h2p_ref, deg_ref, d2p_ref, noise_ref,
                out_ref):
  emb = emb_ref[...]
  fst = fst_ref[...]
  h2 = h2p_ref[0] + h2p_ref[1]
  deg = deg_ref[...]
  d2 = jnp.sum(d2p_ref[...], axis=0)
  scd = (h2 - fst) - deg * emb
  vec = fst + scd
  scdn = (d2 - deg) - deg
  den = (deg + scdn) + jnp.float32(1e-8)
  r = vec / den
  nr = jnp.sqrt(jnp.sum(r * r, axis=1, keepdims=True))
  rn = r / jnp.maximum(nr, jnp.float32(1e-12))
  ne = jnp.sqrt(jnp.sum(emb * emb, axis=1, keepdims=True))
  en = emb / jnp.maximum(ne, jnp.float32(1e-12))
  s = jnp.sum(rn * en, axis=1, keepdims=True)
  out_ref[...] = jnp.log(jax.nn.sigmoid(s)) + noise_ref[...]


def _score(embeds, fst, h2p, deg2d, d2p, noise2d):
  nblk = 25
  rb = _N // nblk
  return pl.pallas_call(
      _score_body,
      grid=(nblk,),
      in_specs=[
          pl.BlockSpec((rb, _D), lambda i: (i, 0)),
          pl.BlockSpec((rb, _D), lambda i: (i, 0)),
          pl.BlockSpec((_NC, rb, _D), lambda i: (0, i, 0)),
          pl.BlockSpec((rb, 1), lambda i: (i, 0)),
          pl.BlockSpec((_NW, rb, 1), lambda i: (0, i, 0)),
          pl.BlockSpec((rb, 1), lambda i: (i, 0)),
      ],
      out_specs=pl.BlockSpec((rb, 1), lambda i: (i, 0)),
      out_shape=jax.ShapeDtypeStruct((_N, 1), jnp.float32),
  )(embeds, fst, h2p, deg2d, d2p, noise2d)


def _rank_body(sc_col_ref, sc_row_ref, seeds_ref):
  sc_row = sc_row_ref[...]
  irow = lax.broadcasted_iota(jnp.int32, (1, _N), 1)
  jb = 400

  def step(c, rank_row):
    sj = sc_col_ref[pl.ds(c * jb, jb), :]
    jcol = c * jb + lax.broadcasted_iota(jnp.int32, (jb, 1), 0)
    gt = (sj > sc_row).astype(jnp.int32)
    tie = ((sj == sc_row) & (jcol < irow)).astype(jnp.int32)
    return rank_row + jnp.sum(gt + tie, axis=0, keepdims=True)

  rank_row = lax.fori_loop(0, _N // jb, step,
                           jnp.zeros((1, _N), jnp.int32))
  pcol = lax.broadcasted_iota(jnp.int32, (_K, 1), 0)
  hit = (rank_row == pcol)
  seeds_ref[...] = jnp.sum(jnp.where(hit, irow, 0), axis=1, keepdims=True)


def _rank(sc_col, sc_row):
  return pl.pallas_call(
      _rank_body,
      out_shape=jax.ShapeDtypeStruct((_K, 1), jnp.int32),
  )(sc_col, sc_row)


def kernel(embeds, edge_index):
  f32 = jnp.float32
  embp = jnp.concatenate(
      [embeds, jnp.zeros((_NPAD - _N, _D), f32)], axis=0)
  ones1d = jnp.ones((_NPAD,), f32)
  zx = jnp.zeros((_B, _D), f32)
  zn = jnp.zeros((_NPAD,), f32)

  edge_1d = edge_index.reshape(2 * _E)
  h1p, degp = _spmm(edge_1d, embp, ones1d, zx, zn)
  fst, deg2d = _combine(h1p, degp.reshape(_NW, _NPAD, 1), embp)
  h2p, d2p = _spmm(edge_1d, fst, deg2d.reshape(_NPAD), zx, zn)
  d2p = d2p.reshape(_NW, _NPAD, 1)

  nk = jax.random.fold_in(jax.random.key(0), 123)
  u = jax.random.uniform(nk, (_N,), dtype=f32, minval=1e-12, maxval=1.0)
  noise = -jnp.log(-jnp.log(u))

  scores2d = _score(embeds, fst, h2p, deg2d, d2p, noise.reshape(_N, 1))
  seeds2d = _rank(scores2d, scores2d.reshape(1, _N))
  return (scores2d.reshape(_N), seeds2d.reshape(_K))


# speedup vs baseline: 9.8057x; 1.6962x over previous
import functools

import jax
import jax.numpy as jnp
from jax import lax
from jax.experimental import pallas as pl
from jax.experimental.pallas import tpu as pltpu
from jax.experimental.pallas import tpu_sc as plsc

_N = 10000
_E = 320000
_D = 128
_K = 512
_NPAD = 10240
_NC = 2
_NS = 16
_NW = _NC * _NS
_EPT = _E // _NW
_B = 80
_NCH = _EPT // _B
_RPS = _NPAD // _NS
_W = 16


def _spmm_body(edge_hbm, x_hbm, dtab_hbm, zx_hbm, zn_hbm,
               xp_hbm, wp_hbm,
               dst0_v, src0_v, dst1_v, src1_v, idx_v,
               rows0_v, rows1_v, dtab_v, dacc_v, accx_sh,
               gsem0, gsem1, ssem0, ssem1):
  cid = lax.axis_index("c")
  sid = lax.axis_index("s")
  wid = sid * _NC + cid
  row0 = sid * _RPS
  lanes = lax.iota(jnp.int32, 16)
  dst = (dst0_v, dst1_v)
  src = (src0_v, src1_v)
  rows = (rows0_v, rows1_v)
  gsem = (gsem0, gsem1)
  ssem = (ssem0, ssem1)

  def fill_idx(k):
    for j in range(_B // 16):
      idx_v[pl.ds(j * 16, 16)] = row0 + k * _B + j * 16 + lanes

  pltpu.sync_copy(zx_hbm, rows0_v)
  pltpu.sync_copy(dtab_hbm, dtab_v)
  pltpu.sync_copy(zn_hbm, dacc_v)
  for k in range(_RPS // _B):
    fill_idx(k)
    pltpu.sync_copy(rows0_v, accx_sh.at[idx_v])
  plsc.subcore_barrier()

  e0 = wid * _EPT

  def load_idx(c, p):
    base = e0 + c * _B
    pltpu.sync_copy(edge_hbm.at[pl.ds(base, _B)], dst[p])
    pltpu.sync_copy(edge_hbm.at[pl.ds(_E + base, _B)], src[p])

  def gather(p):
    return pltpu.make_async_copy(x_hbm.at[src[p]], rows[p], gsem[p])

  def scatter(p):
    return pltpu.make_async_copy(rows[p], accx_sh.at[dst[p]], ssem[p])

  def scalar_channel(p):
    for j in range(_B // 16):
      sidx = src[p][pl.ds(j * 16, 16)]
      didx = dst[p][pl.ds(j * 16, 16)]
      dvals = plsc.load_gather(dtab_v, [sidx])
      plsc.addupdate_scatter(dacc_v, [didx], dvals)

  def slot(c, p, q, first):
    if not first:
      scatter(q).wait()
    load_idx(c + 1, q)
    gather(p).wait()
    gather(q).start()
    scalar_channel(p)
    scatter(p).start(add=True)

  load_idx(0, 0)
  gather(0).start()
  slot(0, 0, 1, True)

  def pair(g, carry):
    slot(2 * g + 1, 1, 0, False)
    slot(2 * g + 2, 0, 1, False)
    return carry

  lax.fori_loop(0, (_NCH - 3) // 2, pair, 0)
  slot(_NCH - 2, 1, 0, False)
  gather(0).wait()
  scalar_channel(0)
  scatter(0).start(add=True)
  scatter(1).wait()
  scatter(0).wait()
  plsc.subcore_barrier()

  for k in range(_RPS // _B):
    fill_idx(k)
    pltpu.sync_copy(accx_sh.at[idx_v], rows0_v)
    pltpu.sync_copy(rows0_v, xp_hbm.at[cid, pl.ds(row0 + k * _B, _B)])
  pltpu.sync_copy(dacc_v, wp_hbm.at[pl.ds(wid * _NPAD, _NPAD)])


@functools.cache
def _spmm_kernel():
  mesh = plsc.VectorSubcoreMesh(
      core_axis_name="c", subcore_axis_name="s",
      num_cores=_NC, num_subcores=_NS)
  return pl.kernel(
      _spmm_body,
      out_type=(
          jax.ShapeDtypeStruct((_NC, _NPAD, _D), jnp.float32),
          jax.ShapeDtypeStruct((_NW * _NPAD,), jnp.float32),
      ),
      mesh=mesh,
      compiler_params=pltpu.CompilerParams(needs_layout_passes=False),
      scratch_types=[
          pltpu.VMEM((_B,), jnp.int32),
          pltpu.VMEM((_B,), jnp.int32),
          pltpu.VMEM((_B,), jnp.int32),
          pltpu.VMEM((_B,), jnp.int32),
          pltpu.VMEM((_B,), jnp.int32),
          pltpu.VMEM((_B, _D), jnp.float32),
          pltpu.VMEM((_B, _D), jnp.float32),
          pltpu.VMEM((_NPAD,), jnp.float32),
          pltpu.VMEM((_NPAD,), jnp.float32),
          pltpu.VMEM_SHARED((_NPAD, _D), jnp.float32),
          pltpu.SemaphoreType.DMA,
          pltpu.SemaphoreType.DMA,
          pltpu.SemaphoreType.DMA,
          pltpu.SemaphoreType.DMA,
      ],
  )


def _spmm(edge_1d, x, dtab, zx, zw):
  return _spmm_kernel()(edge_1d, x, dtab, zx, zw)


def _combine_body(h1p_ref, embp_ref, fst_ref):
  fst_ref[...] = h1p_ref[0] + h1p_ref[1] - embp_ref[...]


def _combine(h1p, embp):
  nblk = 4
  rb = _NPAD // nblk
  return pl.pallas_call(
      _combine_body,
      grid=(nblk,),
      in_specs=[
          pl.BlockSpec((_NC, rb, _D), lambda i: (0, i, 0)),
          pl.BlockSpec((rb, _D), lambda i: (i, 0)),
      ],
      out_specs=pl.BlockSpec((rb, _D), lambda i: (i, 0)),
      out_shape=jax.ShapeDtypeStruct((_NPAD, _D), jnp.float32),
  )(h1p, embp)


def _reduce32_body(p_ref, o_ref):
  o_ref[...] = jnp.sum(p_ref[...], axis=0)


def _reduce32(parts_folded):
  return pl.pallas_call(
      _reduce32_body,
      out_shape=jax.ShapeDtypeStruct((_NPAD // _D, _D), jnp.float32),
  )(parts_folded)


def _score_body(emb_ref, fst_ref, h2p_ref, deg_ref, d2_ref, noise_ref,
                out_ref):
  emb = emb_ref[...]
  fst = fst_ref[...]
  h2 = h2p_ref[0] + h2p_ref[1]
  deg = deg_ref[...]
  d2 = d2_ref[...]
  scd = (h2 - fst) - deg * emb
  vec = fst + scd
  scdn = (d2 - deg) - deg
  den = (deg + scdn) + jnp.float32(1e-8)
  r = vec / den
  nr = jnp.sqrt(jnp.sum(r * r, axis=1, keepdims=True))
  rn = r / jnp.maximum(nr, jnp.float32(1e-12))
  ne = jnp.sqrt(jnp.sum(emb * emb, axis=1, keepdims=True))
  en = emb / jnp.maximum(ne, jnp.float32(1e-12))
  s = jnp.sum(rn * en, axis=1, keepdims=True)
  out_ref[...] = jnp.log(jax.nn.sigmoid(s)) + noise_ref[...]


def _score(embeds, fst, h2p, deg2d, d22d, noise2d):
  nblk = 5
  rb = _N // nblk
  return pl.pallas_call(
      _score_body,
      grid=(nblk,),
      in_specs=[
          pl.BlockSpec((rb, _D), lambda i: (i, 0)),
          pl.BlockSpec((rb, _D), lambda i: (i, 0)),
          pl.BlockSpec((_NC, rb, _D), lambda i: (0, i, 0)),
          pl.BlockSpec((rb, 1), lambda i: (i, 0)),
          pl.BlockSpec((rb, 1), lambda i: (i, 0)),
          pl.BlockSpec((rb, 1), lambda i: (i, 0)),
      ],
      out_specs=pl.BlockSpec((rb, 1), lambda i: (i, 0)),
      out_shape=jax.ShapeDtypeStruct((_N, 1), jnp.float32),
  )(embeds, fst, h2p, deg2d, d22d, noise2d)


def _rank_body(sc_col_ref, sc_row_ref, seeds_ref):
  sc_row = sc_row_ref[...]
  irow = lax.broadcasted_iota(jnp.int32, (1, _N), 1)
  jb = 400

  def step(c, rank_row):
    sj = sc_col_ref[pl.ds(c * jb, jb), :]
    jcol = c * jb + lax.broadcasted_iota(jnp.int32, (jb, 1), 0)
    gt = (sj > sc_row).astype(jnp.int32)
    tie = ((sj == sc_row) & (jcol < irow)).astype(jnp.int32)
    return rank_row + jnp.sum(gt + tie, axis=0, keepdims=True)

  rank_row = lax.fori_loop(0, _N // jb, step,
                           jnp.zeros((1, _N), jnp.int32))
  pcol = lax.broadcasted_iota(jnp.int32, (_K, 1), 0)
  hit = (rank_row == pcol)
  seeds_ref[...] = jnp.sum(jnp.where(hit, irow, 0), axis=1, keepdims=True)


def _rank(sc_col, sc_row):
  return pl.pallas_call(
      _rank_body,
      out_shape=jax.ShapeDtypeStruct((_K, 1), jnp.int32),
  )(sc_col, sc_row)


def kernel(embeds, edge_index):
  f32 = jnp.float32
  embp = jnp.concatenate(
      [embeds, jnp.zeros((_NPAD - _N, _D), f32)], axis=0)
  ones1d = jnp.ones((_NPAD,), f32)
  zx = jnp.zeros((_B, _D), f32)
  zn = jnp.zeros((_NPAD,), f32)

  edge_1d = edge_index.reshape(2 * _E)
  h1p, degp = _spmm(edge_1d, embp, ones1d, zx, zn)
  fst = _combine(h1p, embp)
  deg1d = _reduce32(degp.reshape(_NW, _NPAD // _D, _D)).reshape(_NPAD)
  h2p, d2p = _spmm(edge_1d, fst, deg1d, zx, zn)
  d22d = _reduce32(d2p.reshape(_NW, _NPAD // _D, _D)).reshape(_NPAD, 1)

  nk = jax.random.fold_in(jax.random.key(0), 123)
  u = jax.random.uniform(nk, (_N,), dtype=f32, minval=1e-12, maxval=1.0)
  noise = -jnp.log(-jnp.log(u))

  scores2d = _score(embeds, fst, h2p, deg1d.reshape(_NPAD, 1), d22d,
                    noise.reshape(_N, 1))
  seeds2d = _rank(scores2d, scores2d.reshape(1, _N))
  return (scores2d.reshape(_N), seeds2d.reshape(_K))


# speedup vs baseline: 12.7267x; 1.2979x over previous
import functools

import jax
import jax.numpy as jnp
from jax import lax
from jax.experimental import pallas as pl
from jax.experimental.pallas import tpu as pltpu
from jax.experimental.pallas import tpu_sc as plsc

_N = 10000
_E = 320000
_D = 128
_K = 512
_NPAD = 10240
_NC = 2
_NS = 16
_NW = _NC * _NS
_EPT = _E // _NW
_B = 80
_NCH = _EPT // _B
_RPS = _NPAD // _NS
_W = 16


def _spmm_body(edge_hbm, x_hbm, dtab_hbm, zx_hbm, zn_hbm,
               xp_hbm, wp_hbm,
               dst0_v, src0_v, dst1_v, src1_v, dst2_v, src2_v, idx_v,
               rows0_v, rows1_v, dtab_v, dacc_v, accx_sh,
               gsem0, gsem1, ssem0, ssem1,
               isem0, isem1, isem2):
  cid = lax.axis_index("c")
  sid = lax.axis_index("s")
  wid = sid * _NC + cid
  row0 = sid * _RPS
  lanes = lax.iota(jnp.int32, 16)
  dst = (dst0_v, dst1_v, dst2_v)
  src = (src0_v, src1_v, src2_v)
  rows = (rows0_v, rows1_v)
  gsem = (gsem0, gsem1)
  ssem = (ssem0, ssem1)
  isem = (isem0, isem1, isem2)

  def fill_idx(k):
    for j in range(_B // 16):
      idx_v[pl.ds(j * 16, 16)] = row0 + k * _B + j * 16 + lanes

  pltpu.sync_copy(zx_hbm, rows0_v)
  pltpu.sync_copy(dtab_hbm, dtab_v)
  pltpu.sync_copy(zn_hbm, dacc_v)
  for k in range(_RPS // _B):
    fill_idx(k)
    pltpu.sync_copy(rows0_v, accx_sh.at[idx_v])
  plsc.subcore_barrier()

  e0 = wid * _EPT

  def idx_copies(c, r):
    base = e0 + c * _B
    return (pltpu.make_async_copy(edge_hbm.at[pl.ds(base, _B)],
                                  dst[r], isem[r]),
            pltpu.make_async_copy(edge_hbm.at[pl.ds(_E + base, _B)],
                                  src[r], isem[r]))

  def start_idx(c, r):
    a, b = idx_copies(c, r)
    a.start()
    b.start()

  def wait_idx(c, r):
    a, b = idx_copies(c, r)
    a.wait()
    b.wait()

  def gather(pr, r):
    return pltpu.make_async_copy(x_hbm.at[src[r]], rows[pr], gsem[pr])

  def scatter(pr, r):
    return pltpu.make_async_copy(rows[pr], accx_sh.at[dst[r]], ssem[pr])

  def scalar_channel(r):
    for j in range(_B // 16):
      sidx = src[r][pl.ds(j * 16, 16)]
      didx = dst[r][pl.ds(j * 16, 16)]
      dvals = plsc.load_gather(dtab_v, [sidx])
      plsc.addupdate_scatter(dacc_v, [didx], dvals)

  def slot(pos, cval, first=False, prefetch=True):
    r = pos % 3
    pr = pos % 2
    r1 = (r + 1) % 3
    r2 = (r + 2) % 3
    rm1 = (r + 2) % 3
    wait_idx(cval + 1, r1)
    if not first:
      scatter(1 - pr, rm1).wait()
    if prefetch:
      start_idx(cval + 2, r2)
    gather(1 - pr, r1).start()
    gather(pr, r).wait()
    scalar_channel(r)
    scatter(pr, r).start(add=True)

  start_idx(0, 0)
  start_idx(1, 1)
  wait_idx(0, 0)
  gather(0, 0).start()
  slot(0, 0, first=True)

  def ring(g, carry):
    c0 = 6 * g + 1
    for k in range(6):
      slot(k + 1, c0 + k)
    return carry

  lax.fori_loop(0, 20, ring, 0)
  slot(121, 121)
  slot(122, 122)
  slot(123, 123, prefetch=False)
  gather(0, 1).wait()
  scalar_channel(1)
  scatter(0, 1).start(add=True)
  scatter(1, 0).wait()
  scatter(0, 1).wait()
  plsc.subcore_barrier()

  for k in range(_RPS // _B):
    fill_idx(k)
    pltpu.sync_copy(accx_sh.at[idx_v], rows0_v)
    pltpu.sync_copy(rows0_v, xp_hbm.at[cid, pl.ds(row0 + k * _B, _B)])
  pltpu.sync_copy(dacc_v, wp_hbm.at[pl.ds(wid * _NPAD, _NPAD)])


@functools.cache
def _spmm_kernel():
  mesh = plsc.VectorSubcoreMesh(
      core_axis_name="c", subcore_axis_name="s",
      num_cores=_NC, num_subcores=_NS)
  return pl.kernel(
      _spmm_body,
      out_type=(
          jax.ShapeDtypeStruct((_NC, _NPAD, _D), jnp.float32),
          jax.ShapeDtypeStruct((_NW * _NPAD,), jnp.float32),
      ),
      mesh=mesh,
      compiler_params=pltpu.CompilerParams(needs_layout_passes=False),
      scratch_types=[
          pltpu.VMEM((_B,), jnp.int32),
          pltpu.VMEM((_B,), jnp.int32),
          pltpu.VMEM((_B,), jnp.int32),
          pltpu.VMEM((_B,), jnp.int32),
          pltpu.VMEM((_B,), jnp.int32),
          pltpu.VMEM((_B,), jnp.int32),
          pltpu.VMEM((_B,), jnp.int32),
          pltpu.VMEM((_B, _D), jnp.float32),
          pltpu.VMEM((_B, _D), jnp.float32),
          pltpu.VMEM((_NPAD,), jnp.float32),
          pltpu.VMEM((_NPAD,), jnp.float32),
          pltpu.VMEM_SHARED((_NPAD, _D), jnp.float32),
          pltpu.SemaphoreType.DMA,
          pltpu.SemaphoreType.DMA,
          pltpu.SemaphoreType.DMA,
          pltpu.SemaphoreType.DMA,
          pltpu.SemaphoreType.DMA,
          pltpu.SemaphoreType.DMA,
          pltpu.SemaphoreType.DMA,
      ],
  )


def _spmm(edge_1d, x, dtab, zx, zw):
  return _spmm_kernel()(edge_1d, x, dtab, zx, zw)


def _combine_body(h1p_ref, embp_ref, fst_ref):
  fst_ref[...] = h1p_ref[0] + h1p_ref[1] - embp_ref[...]


def _combine(h1p, embp):
  nblk = 4
  rb = _NPAD // nblk
  return pl.pallas_call(
      _combine_body,
      grid=(nblk,),
      in_specs=[
          pl.BlockSpec((_NC, rb, _D), lambda i: (0, i, 0)),
          pl.BlockSpec((rb, _D), lambda i: (i, 0)),
      ],
      out_specs=pl.BlockSpec((rb, _D), lambda i: (i, 0)),
      out_shape=jax.ShapeDtypeStruct((_NPAD, _D), jnp.float32),
  )(h1p, embp)


def _reduce32_body(p_ref, o_ref):
  o_ref[...] = jnp.sum(p_ref[...], axis=0)


def _reduce32(parts_folded):
  return pl.pallas_call(
      _reduce32_body,
      out_shape=jax.ShapeDtypeStruct((_NPAD // _D, _D), jnp.float32),
  )(parts_folded)


def _score_body(emb_ref, fst_ref, h2p_ref, deg_ref, d2_ref, noise_ref,
                out_ref):
  emb = emb_ref[...]
  fst = fst_ref[...]
  h2 = h2p_ref[0] + h2p_ref[1]
  deg = deg_ref[...]
  d2 = d2_ref[...]
  scd = (h2 - fst) - deg * emb
  vec = fst + scd
  scdn = (d2 - deg) - deg
  den = (deg + scdn) + jnp.float32(1e-8)
  r = vec / den
  nr = jnp.sqrt(jnp.sum(r * r, axis=1, keepdims=True))
  rn = r / jnp.maximum(nr, jnp.float32(1e-12))
  ne = jnp.sqrt(jnp.sum(emb * emb, axis=1, keepdims=True))
  en = emb / jnp.maximum(ne, jnp.float32(1e-12))
  s = jnp.sum(rn * en, axis=1, keepdims=True)
  out_ref[...] = jnp.log(jax.nn.sigmoid(s)) + noise_ref[...]


def _score(embeds, fst, h2p, deg2d, d22d, noise2d):
  nblk = 5
  rb = _N // nblk
  return pl.pallas_call(
      _score_body,
      grid=(nblk,),
      in_specs=[
          pl.BlockSpec((rb, _D), lambda i: (i, 0)),
          pl.BlockSpec((rb, _D), lambda i: (i, 0)),
          pl.BlockSpec((_NC, rb, _D), lambda i: (0, i, 0)),
          pl.BlockSpec((rb, 1), lambda i: (i, 0)),
          pl.BlockSpec((rb, 1), lambda i: (i, 0)),
          pl.BlockSpec((rb, 1), lambda i: (i, 0)),
      ],
      out_specs=pl.BlockSpec((rb, 1), lambda i: (i, 0)),
      out_shape=jax.ShapeDtypeStruct((_N, 1), jnp.float32),
  )(embeds, fst, h2p, deg2d, d22d, noise2d)


def _rank_body(sc_col_ref, sc_row_ref, seeds_ref):
  sc_row = sc_row_ref[...]
  irow = lax.broadcasted_iota(jnp.int32, (1, _N), 1)
  jb = 400

  def step(c, rank_row):
    sj = sc_col_ref[pl.ds(c * jb, jb), :]
    jcol = c * jb + lax.broadcasted_iota(jnp.int32, (jb, 1), 0)
    gt = (sj > sc_row).astype(jnp.int32)
    tie = ((sj == sc_row) & (jcol < irow)).astype(jnp.int32)
    return rank_row + jnp.sum(gt + tie, axis=0, keepdims=True)

  rank_row = lax.fori_loop(0, _N // jb, step,
                           jnp.zeros((1, _N), jnp.int32))
  pcol = lax.broadcasted_iota(jnp.int32, (_K, 1), 0)
  hit = (rank_row == pcol)
  seeds_ref[...] = jnp.sum(jnp.where(hit, irow, 0), axis=1, keepdims=True)


def _rank(sc_col, sc_row):
  return pl.pallas_call(
      _rank_body,
      out_shape=jax.ShapeDtypeStruct((_K, 1), jnp.int32),
  )(sc_col, sc_row)


def kernel(embeds, edge_index):
  f32 = jnp.float32
  embp = jnp.concatenate(
      [embeds, jnp.zeros((_NPAD - _N, _D), f32)], axis=0)
  ones1d = jnp.ones((_NPAD,), f32)
  zx = jnp.zeros((_B, _D), f32)
  zn = jnp.zeros((_NPAD,), f32)

  edge_1d = edge_index.reshape(2 * _E)
  h1p, degp = _spmm(edge_1d, embp, ones1d, zx, zn)
  fst = _combine(h1p, embp)
  deg1d = _reduce32(degp.reshape(_NW, _NPAD // _D, _D)).reshape(_NPAD)
  h2p, d2p = _spmm(edge_1d, fst, deg1d, zx, zn)
  d22d = _reduce32(d2p.reshape(_NW, _NPAD // _D, _D)).reshape(_NPAD, 1)

  nk = jax.random.fold_in(jax.random.key(0), 123)
  u = jax.random.uniform(nk, (_N,), dtype=f32, minval=1e-12, maxval=1.0)
  noise = -jnp.log(-jnp.log(u))

  scores2d = _score(embeds, fst, h2p, deg1d.reshape(_NPAD, 1), d22d,
                    noise.reshape(_N, 1))
  seeds2d = _rank(scores2d, scores2d.reshape(1, _N))
  return (scores2d.reshape(_N), seeds2d.reshape(_K))
